# baseline (speedup 1.0000x reference)
import functools

import jax
import jax.numpy as jnp
from jax import lax
from jax.experimental import pallas as pl
from jax.experimental.pallas import tpu as pltpu
from jax.experimental.pallas import tpu_sc as plsc

_K = 200
_B, _C, _H, _W = 16, 96, 224, 224
_HW = _H * _W
_NV = _HW // 16
_C_BLK = 48
_J = _C // _C_BLK
_N_OPS = 8


def _diff_kernel(*refs):
    preds = refs[:_N_OPS]
    targs = refs[_N_OPS:2 * _N_OPS]
    out_ref = refs[2 * _N_OPS]
    acc_ref = refs[2 * _N_OPS + 1]
    j = pl.program_id(1)

    part = jnp.zeros((_H, _W), jnp.float32)
    for p_ref, t_ref in zip(preds, targs):
        d = t_ref[0] - p_ref[0]
        part = part + jnp.sum(d * d, axis=0)

    @pl.when(j == 0)
    def _first():
        acc_ref[:, :] = part

    @pl.when(j > 0)
    def _rest():
        acc_ref[:, :] = acc_ref[:, :] + part

    @pl.when(j == _J - 1)
    def _out():
        out_ref[0] = acc_ref[:, :]


def _diff(pred, target, b0, nb):
    sub = _C_BLK // _N_OPS
    specs = [
        pl.BlockSpec((1, sub, _H, _W),
                     lambda b, j, i=i: (b + b0, _N_OPS * j + i, 0, 0))
        for i in range(_N_OPS)
    ]
    return pl.pallas_call(
        _diff_kernel,
        grid=(nb, _J),
        in_specs=specs + specs,
        out_specs=pl.BlockSpec((1, _H, _W), lambda b, j: (b, 0, 0)),
        out_shape=jax.ShapeDtypeStruct((nb, _H, _W), jnp.float32),
        scratch_shapes=[pltpu.VMEM((_H, _W), jnp.float32)],
        compiler_params=pltpu.CompilerParams(
            dimension_semantics=("parallel", "arbitrary")),
    )(*([pred] * _N_OPS + [target] * _N_OPS))


_BITS_HI = 0x7F800001


def _make_sc_topk_kernel(nb):
  def _sc_topk_kernel(diff_hbm, out_hbm, v_ref, outv_ref, redi_ref, redf_ref):
    cid = lax.axis_index("c")
    sid = lax.axis_index("s")

    @pl.when((cid == 0) & (sid < nb))
    def _work():
        b = sid
        pltpu.sync_copy(diff_hbm.at[b], v_ref)

        ones_i = jnp.ones((16,), jnp.int32)
        zero_i = jnp.zeros((16,), jnp.int32)
        zero_f = jnp.zeros((16,), jnp.float32)
        k_vec = jnp.full((16,), _K, jnp.int32)

        def reduce_full(vec, buf_ref):
            buf_ref[pl.ds(0, 16)] = vec
            buf_ref[pl.ds(16, 16)] = vec
            for sh in (8, 4, 2, 1):
                r = buf_ref[pl.ds(0, 16)] + buf_ref[pl.ds(sh, 16)]
                buf_ref[pl.ds(0, 16)] = r
                buf_ref[pl.ds(16, 16)] = r
            return buf_ref[pl.ds(0, 16)]

        def count_ge(mid):
            def cbody(i, cnt):
                base = i * 64
                for u in range(4):
                    x = v_ref[pl.ds(base + u * 16, 16)]
                    xi = lax.bitcast_convert_type(x, jnp.int32)
                    cnt = cnt + jnp.where(xi >= mid, ones_i, zero_i)
                return cnt
            return reduce_full(lax.fori_loop(0, _NV // 4, cbody, zero_i),
                               redi_ref)

        def sbody(_, carry):
            lo, hi = carry
            mid = lo + lax.shift_right_arithmetic(hi - lo, 1)
            ge = count_ge(mid) >= k_vec
            return jnp.where(ge, mid, lo), jnp.where(ge, hi, mid)

        t_bits, _hi = lax.fori_loop(
            0, 31, sbody,
            (zero_i, jnp.full((16,), _BITS_HI, jnp.int32)))

        def fbody(i, carry):
            s_vec, c_vec = carry
            base = i * 64
            for u in range(4):
                x = v_ref[pl.ds(base + u * 16, 16)]
                xi = lax.bitcast_convert_type(x, jnp.int32)
                gt = xi > t_bits
                s_vec = s_vec + jnp.where(gt, x, zero_f)
                c_vec = c_vec + jnp.where(gt, ones_i, zero_i)
            return (s_vec, c_vec)

        s_vec, c_vec = lax.fori_loop(
            0, _NV // 4, fbody, (zero_f, zero_i))
        s_tot = reduce_full(s_vec, redf_ref)
        c_tot = reduce_full(c_vec, redi_ref)
        t_val = lax.bitcast_convert_type(t_bits, jnp.float32)
        outv_ref[...] = s_tot + (k_vec - c_tot).astype(jnp.float32) * t_val
        pltpu.sync_copy(outv_ref, out_hbm.at[b])

  return _sc_topk_kernel


def _sc_topk(diff2, nb):
    mesh = plsc.VectorSubcoreMesh(core_axis_name="c", subcore_axis_name="s")
    return functools.partial(
        pl.kernel, mesh=mesh,
        out_type=jax.ShapeDtypeStruct((nb, 16), jnp.float32),
        scratch_types=[
            pltpu.VMEM((_HW,), jnp.float32),
            pltpu.VMEM((16,), jnp.float32),
            pltpu.VMEM((32,), jnp.int32),
            pltpu.VMEM((32,), jnp.float32),
        ],
    )(_make_sc_topk_kernel(nb))(diff2)


def kernel(pred, target):
    nh = _B // 2
    d0 = _diff(pred, target, 0, nh)
    s0 = _sc_topk(d0.reshape(nh, _HW), nh)
    d1 = _diff(pred, target, nh, nh)
    s1 = _sc_topk(d1.reshape(nh, _HW), nh)
    return (jnp.sum(s0[:, 0]) + jnp.sum(s1[:, 0])) / (_B * _K)


# speedup vs baseline: 1.0611x; 1.0611x over previous
import functools

import jax
import jax.numpy as jnp
from jax import lax
from jax.experimental import pallas as pl
from jax.experimental.pallas import tpu as pltpu
from jax.experimental.pallas import tpu_sc as plsc

_K = 200
_B, _C, _H, _W = 16, 96, 224, 224
_HW = _H * _W
_NV = _HW // 16
_C_BLK = 48
_J = _C // _C_BLK
_N_OPS = 8


def _diff_kernel(*refs):
    preds = refs[:_N_OPS]
    targs = refs[_N_OPS:2 * _N_OPS]
    out_ref = refs[2 * _N_OPS]
    acc_ref = refs[2 * _N_OPS + 1]
    j = pl.program_id(1)

    part = jnp.zeros((_H, _W), jnp.float32)
    for p_ref, t_ref in zip(preds, targs):
        d = t_ref[0] - p_ref[0]
        part = part + jnp.sum(d * d, axis=0)

    @pl.when(j == 0)
    def _first():
        acc_ref[:, :] = part

    @pl.when(j > 0)
    def _rest():
        acc_ref[:, :] = acc_ref[:, :] + part

    @pl.when(j == _J - 1)
    def _out():
        out_ref[0] = acc_ref[:, :]


def _diff(pred, target):
    sub = _C_BLK // _N_OPS
    specs = [
        pl.BlockSpec((1, sub, _H, _W),
                     lambda b, j, i=i: (b, _N_OPS * j + i, 0, 0))
        for i in range(_N_OPS)
    ]
    return pl.pallas_call(
        _diff_kernel,
        grid=(_B, _J),
        in_specs=specs + specs,
        out_specs=pl.BlockSpec((1, _H, _W), lambda b, j: (b, 0, 0)),
        out_shape=jax.ShapeDtypeStruct((_B, _H, _W), jnp.float32),
        scratch_shapes=[pltpu.VMEM((_H, _W), jnp.float32)],
        compiler_params=pltpu.CompilerParams(
            dimension_semantics=("parallel", "arbitrary")),
    )(*([pred] * _N_OPS + [target] * _N_OPS))


_BITS_HI = 0x7F800001


def _sc_topk_kernel(diff_hbm, out_hbm, v_ref, outv_ref, redi_ref, redf_ref):
    cid = lax.axis_index("c")
    sid = lax.axis_index("s")

    @pl.when(cid == 0)
    def _work():
        b = sid
        pltpu.sync_copy(diff_hbm.at[b], v_ref)

        ones_i = jnp.ones((16,), jnp.int32)
        zero_i = jnp.zeros((16,), jnp.int32)
        zero_f = jnp.zeros((16,), jnp.float32)
        k_vec = jnp.full((16,), _K, jnp.int32)

        def reduce_full(vec, buf_ref):
            buf_ref[pl.ds(0, 16)] = vec
            buf_ref[pl.ds(16, 16)] = vec
            for sh in (8, 4, 2, 1):
                r = buf_ref[pl.ds(0, 16)] + buf_ref[pl.ds(sh, 16)]
                buf_ref[pl.ds(0, 16)] = r
                buf_ref[pl.ds(16, 16)] = r
            return buf_ref[pl.ds(0, 16)]

        def count_ge(mid):
            def cbody(i, cnt):
                base = i * 64
                for u in range(4):
                    x = v_ref[pl.ds(base + u * 16, 16)]
                    xi = lax.bitcast_convert_type(x, jnp.int32)
                    cnt = cnt + jnp.where(xi >= mid, ones_i, zero_i)
                return cnt
            return reduce_full(lax.fori_loop(0, _NV // 4, cbody, zero_i),
                               redi_ref)

        def sbody(_, carry):
            lo, hi = carry
            mid = lo + lax.shift_right_arithmetic(hi - lo, 1)
            ge = count_ge(mid) >= k_vec
            return jnp.where(ge, mid, lo), jnp.where(ge, hi, mid)

        t_bits, _hi = lax.fori_loop(
            0, 31, sbody,
            (zero_i, jnp.full((16,), _BITS_HI, jnp.int32)))

        def fbody(i, carry):
            s_vec, c_vec = carry
            base = i * 64
            for u in range(4):
                x = v_ref[pl.ds(base + u * 16, 16)]
                xi = lax.bitcast_convert_type(x, jnp.int32)
                gt = xi > t_bits
                s_vec = s_vec + jnp.where(gt, x, zero_f)
                c_vec = c_vec + jnp.where(gt, ones_i, zero_i)
            return (s_vec, c_vec)

        s_vec, c_vec = lax.fori_loop(
            0, _NV // 4, fbody, (zero_f, zero_i))
        s_tot = reduce_full(s_vec, redf_ref)
        c_tot = reduce_full(c_vec, redi_ref)
        t_val = lax.bitcast_convert_type(t_bits, jnp.float32)
        outv_ref[...] = s_tot + (k_vec - c_tot).astype(jnp.float32) * t_val
        pltpu.sync_copy(outv_ref, out_hbm.at[b])


def _sc_topk(diff2):
    mesh = plsc.VectorSubcoreMesh(core_axis_name="c", subcore_axis_name="s")
    return functools.partial(
        pl.kernel, mesh=mesh,
        out_type=jax.ShapeDtypeStruct((_B, 16), jnp.float32),
        scratch_types=[
            pltpu.VMEM((_HW,), jnp.float32),
            pltpu.VMEM((16,), jnp.float32),
            pltpu.VMEM((32,), jnp.int32),
            pltpu.VMEM((32,), jnp.float32),
        ],
    )(_sc_topk_kernel)(diff2)


def kernel(pred, target):
    diff = _diff(pred, target)
    sums = _sc_topk(diff.reshape(_B, _HW))
    return jnp.sum(sums[:, 0]) / (_B * _K)


# speedup vs baseline: 1.1592x; 1.0925x over previous
import functools

import jax
import jax.numpy as jnp
from jax import lax
from jax.experimental import pallas as pl
from jax.experimental.pallas import tpu as pltpu
from jax.experimental.pallas import tpu_sc as plsc

_K = 200
_B, _C, _H, _W = 16, 96, 224, 224
_HW = _H * _W
_NV = _HW // 16
_C_BLK = 48
_J = _C // _C_BLK
_N_OPS = 8


def _diff_kernel(*refs):
    preds = refs[:_N_OPS]
    targs = refs[_N_OPS:2 * _N_OPS]
    out_ref = refs[2 * _N_OPS]
    acc_ref = refs[2 * _N_OPS + 1]
    j = pl.program_id(1)

    part = jnp.zeros((_H, _W), jnp.float32)
    for p_ref, t_ref in zip(preds, targs):
        d = t_ref[0] - p_ref[0]
        part = part + jnp.sum(d * d, axis=0)

    @pl.when(j == 0)
    def _first():
        acc_ref[:, :] = part

    @pl.when(j > 0)
    def _rest():
        acc_ref[:, :] = acc_ref[:, :] + part

    @pl.when(j == _J - 1)
    def _out():
        out_ref[0] = acc_ref[:, :]


def _diff(pred, target):
    sub = _C_BLK // _N_OPS
    specs = [
        pl.BlockSpec((1, sub, _H, _W),
                     lambda b, j, i=i: (b, _N_OPS * j + i, 0, 0))
        for i in range(_N_OPS)
    ]
    return pl.pallas_call(
        _diff_kernel,
        grid=(_B, _J),
        in_specs=specs + specs,
        out_specs=pl.BlockSpec((1, _H, _W), lambda b, j: (b, 0, 0)),
        out_shape=jax.ShapeDtypeStruct((_B, _H, _W), jnp.float32),
        scratch_shapes=[pltpu.VMEM((_H, _W), jnp.float32)],
        compiler_params=pltpu.CompilerParams(
            dimension_semantics=("parallel", "arbitrary")),
    )(*([pred] * _N_OPS + [target] * _N_OPS))


_BITS_HI = 0x7F800001


def _sc_topk_kernel(diff_hbm, out_hbm, v_ref, outv_ref, redi_ref, redf_ref):
    cid = lax.axis_index("c")
    sid = lax.axis_index("s")

    @pl.when(cid == 0)
    def _work():
        b = sid
        pltpu.sync_copy(diff_hbm.at[b], v_ref)

        ones_i = jnp.ones((16,), jnp.int32)
        zero_i = jnp.zeros((16,), jnp.int32)
        zero_f = jnp.zeros((16,), jnp.float32)
        k_vec = jnp.full((16,), _K, jnp.int32)

        def reduce_full(vec, buf_ref):
            buf_ref[pl.ds(0, 16)] = vec
            buf_ref[pl.ds(16, 16)] = vec
            for sh in (8, 4, 2, 1):
                r = buf_ref[pl.ds(0, 16)] + buf_ref[pl.ds(sh, 16)]
                buf_ref[pl.ds(0, 16)] = r
                buf_ref[pl.ds(16, 16)] = r
            return buf_ref[pl.ds(0, 16)]

        def count_ge(mid):
            def cbody(i, cnt):
                base = i * 128
                for u in range(8):
                    x = v_ref[pl.ds(base + u * 16, 16)]
                    xi = lax.bitcast_convert_type(x, jnp.int32)
                    cnt = cnt + jnp.where(xi >= mid, ones_i, zero_i)
                return cnt
            return reduce_full(lax.fori_loop(0, _NV // 8, cbody, zero_i),
                               redi_ref)

        def sbody(_, carry):
            lo, hi = carry
            mid = lo + lax.shift_right_arithmetic(hi - lo, 1)
            ge = count_ge(mid) >= k_vec
            return jnp.where(ge, mid, lo), jnp.where(ge, hi, mid)

        t_bits, _hi = lax.fori_loop(
            0, 31, sbody,
            (zero_i, jnp.full((16,), _BITS_HI, jnp.int32)))

        def fbody(i, carry):
            s_vec, c_vec = carry
            base = i * 64
            for u in range(4):
                x = v_ref[pl.ds(base + u * 16, 16)]
                xi = lax.bitcast_convert_type(x, jnp.int32)
                gt = xi > t_bits
                s_vec = s_vec + jnp.where(gt, x, zero_f)
                c_vec = c_vec + jnp.where(gt, ones_i, zero_i)
            return (s_vec, c_vec)

        s_vec, c_vec = lax.fori_loop(
            0, _NV // 4, fbody, (zero_f, zero_i))
        s_tot = reduce_full(s_vec, redf_ref)
        c_tot = reduce_full(c_vec, redi_ref)
        t_val = lax.bitcast_convert_type(t_bits, jnp.float32)
        outv_ref[...] = s_tot + (k_vec - c_tot).astype(jnp.float32) * t_val
        pltpu.sync_copy(outv_ref, out_hbm.at[b])


def _sc_topk(diff2):
    mesh = plsc.VectorSubcoreMesh(core_axis_name="c", subcore_axis_name="s")
    return functools.partial(
        pl.kernel, mesh=mesh,
        out_type=jax.ShapeDtypeStruct((_B, 16), jnp.float32),
        scratch_types=[
            pltpu.VMEM((_HW,), jnp.float32),
            pltpu.VMEM((16,), jnp.float32),
            pltpu.VMEM((32,), jnp.int32),
            pltpu.VMEM((32,), jnp.float32),
        ],
    )(_sc_topk_kernel)(diff2)


def kernel(pred, target):
    diff = _diff(pred, target)
    sums = _sc_topk(diff.reshape(_B, _HW))
    return jnp.sum(sums[:, 0]) / (_B * _K)


# speedup vs baseline: 1.1597x; 1.0005x over previous
import functools

import jax
import jax.numpy as jnp
from jax import lax
from jax.experimental import pallas as pl
from jax.experimental.pallas import tpu as pltpu
from jax.experimental.pallas import tpu_sc as plsc

_K = 200
_B, _C, _H, _W = 16, 96, 224, 224
_HW = _H * _W
_NV = _HW // 16
_C_BLK = 48
_J = _C // _C_BLK
_N_OPS = 8


def _diff_kernel(*refs):
    preds = refs[:_N_OPS]
    targs = refs[_N_OPS:2 * _N_OPS]
    out_ref = refs[2 * _N_OPS]
    acc_ref = refs[2 * _N_OPS + 1]
    j = pl.program_id(1)

    part = jnp.zeros((_H, _W), jnp.float32)
    for p_ref, t_ref in zip(preds, targs):
        d = t_ref[0] - p_ref[0]
        part = part + jnp.sum(d * d, axis=0)

    @pl.when(j == 0)
    def _first():
        acc_ref[:, :] = part

    @pl.when(j > 0)
    def _rest():
        acc_ref[:, :] = acc_ref[:, :] + part

    @pl.when(j == _J - 1)
    def _out():
        out_ref[0] = acc_ref[:, :]


def _diff(pred, target):
    sub = _C_BLK // _N_OPS
    specs = [
        pl.BlockSpec((1, sub, _H, _W),
                     lambda b, j, i=i: (b, _N_OPS * j + i, 0, 0))
        for i in range(_N_OPS)
    ]
    return pl.pallas_call(
        _diff_kernel,
        grid=(_B, _J),
        in_specs=specs + specs,
        out_specs=pl.BlockSpec((1, _H, _W), lambda b, j: (b, 0, 0)),
        out_shape=jax.ShapeDtypeStruct((_B, _H, _W), jnp.float32),
        scratch_shapes=[pltpu.VMEM((_H, _W), jnp.float32)],
        compiler_params=pltpu.CompilerParams(
            dimension_semantics=("parallel", "arbitrary")),
    )(*([pred] * _N_OPS + [target] * _N_OPS))


_BITS_HI = 0x7F800001


def _sc_topk_kernel(diff_hbm, out_hbm, v_ref, outv_ref, redi_ref, redf_ref):
    cid = lax.axis_index("c")
    sid = lax.axis_index("s")

    @pl.when(cid == 0)
    def _work():
        b = sid
        pltpu.sync_copy(diff_hbm.at[b], v_ref)

        ones_i = jnp.ones((16,), jnp.int32)
        zero_i = jnp.zeros((16,), jnp.int32)
        zero_f = jnp.zeros((16,), jnp.float32)
        k_vec = jnp.full((16,), _K, jnp.int32)

        def reduce_full(vec, buf_ref):
            buf_ref[pl.ds(0, 16)] = vec
            buf_ref[pl.ds(16, 16)] = vec
            for sh in (8, 4, 2, 1):
                r = buf_ref[pl.ds(0, 16)] + buf_ref[pl.ds(sh, 16)]
                buf_ref[pl.ds(0, 16)] = r
                buf_ref[pl.ds(16, 16)] = r
            return buf_ref[pl.ds(0, 16)]

        def count_ge(mid):
            def cbody(i, cnt):
                base = i * 256
                for u in range(16):
                    x = v_ref[pl.ds(base + u * 16, 16)]
                    xi = lax.bitcast_convert_type(x, jnp.int32)
                    cnt = cnt + jnp.where(xi >= mid, ones_i, zero_i)
                return cnt
            return reduce_full(lax.fori_loop(0, _NV // 16, cbody, zero_i),
                               redi_ref)

        def sbody(_, carry):
            lo, hi = carry
            mid = lo + lax.shift_right_arithmetic(hi - lo, 1)
            ge = count_ge(mid) >= k_vec
            return jnp.where(ge, mid, lo), jnp.where(ge, hi, mid)

        t_bits, _hi = lax.fori_loop(
            0, 31, sbody,
            (zero_i, jnp.full((16,), _BITS_HI, jnp.int32)))

        def fbody(i, carry):
            s_vec, c_vec = carry
            base = i * 64
            for u in range(4):
                x = v_ref[pl.ds(base + u * 16, 16)]
                xi = lax.bitcast_convert_type(x, jnp.int32)
                gt = xi > t_bits
                s_vec = s_vec + jnp.where(gt, x, zero_f)
                c_vec = c_vec + jnp.where(gt, ones_i, zero_i)
            return (s_vec, c_vec)

        s_vec, c_vec = lax.fori_loop(
            0, _NV // 4, fbody, (zero_f, zero_i))
        s_tot = reduce_full(s_vec, redf_ref)
        c_tot = reduce_full(c_vec, redi_ref)
        t_val = lax.bitcast_convert_type(t_bits, jnp.float32)
        outv_ref[...] = s_tot + (k_vec - c_tot).astype(jnp.float32) * t_val
        pltpu.sync_copy(outv_ref, out_hbm.at[b])


def _sc_topk(diff2):
    mesh = plsc.VectorSubcoreMesh(core_axis_name="c", subcore_axis_name="s")
    return functools.partial(
        pl.kernel, mesh=mesh,
        out_type=jax.ShapeDtypeStruct((_B, 16), jnp.float32),
        scratch_types=[
            pltpu.VMEM((_HW,), jnp.float32),
            pltpu.VMEM((16,), jnp.float32),
            pltpu.VMEM((32,), jnp.int32),
            pltpu.VMEM((32,), jnp.float32),
        ],
    )(_sc_topk_kernel)(diff2)


def kernel(pred, target):
    diff = _diff(pred, target)
    sums = _sc_topk(diff.reshape(_B, _HW))
    return jnp.sum(sums[:, 0]) / (_B * _K)
